# baseline (speedup 1.0000x reference)
import functools

import jax
import jax.numpy as jnp
from jax import lax
from jax.experimental import pallas as pl
from jax.experimental.pallas import tpu as pltpu

NPTS = 10000
FEAT = 128
KNB = 16
DILS = (1, 2)
UPR = 4
NBLOCKS = 2
NLAYERS = 2
KSEL = KNB * max(DILS) + 1

ROWPAD = 10240
COLPAD = 10112
DIST_ROWBLK = 256
LIN_ROWBLK = 512


def _dist_body(pos_ref, post_ref, out_ref):
    pr = pos_ref[...]
    pt = post_ref[...]
    d2r = jnp.sum(pr * pr, axis=1, keepdims=True)
    d2c = jnp.sum(pt * pt, axis=0, keepdims=True)
    g = jnp.dot(pr, pt, preferred_element_type=jnp.float32)
    d = d2r + d2c - 2.0 * g
    col = lax.broadcasted_iota(jnp.int32, d.shape, 1)
    out_ref[...] = jnp.where(col < NPTS, d, jnp.inf)


def _pairwise_dist(pos_pad):
    post = pos_pad.T[:, :COLPAD]
    return pl.pallas_call(
        _dist_body,
        grid=(ROWPAD // DIST_ROWBLK,),
        in_specs=[
            pl.BlockSpec((DIST_ROWBLK, 8), lambda i: (i, 0)),
            pl.BlockSpec((8, COLPAD), lambda i: (0, 0)),
        ],
        out_specs=pl.BlockSpec((DIST_ROWBLK, COLPAD), lambda i: (i, 0)),
        out_shape=jax.ShapeDtypeStruct((ROWPAD, COLPAD), jnp.float32),
    )(pos_pad, post)


def _lin_body(h_ref, w_ref, b_ref, o_ref):
    o_ref[...] = (
        jnp.dot(h_ref[...], w_ref[...], preferred_element_type=jnp.float32,
                precision=lax.Precision.HIGHEST)
        + b_ref[...]
    )


def _lin(h, w, b):
    cin, cout = w.shape
    return pl.pallas_call(
        _lin_body,
        grid=(h.shape[0] // LIN_ROWBLK,),
        in_specs=[
            pl.BlockSpec((LIN_ROWBLK, cin), lambda i: (i, 0)),
            pl.BlockSpec((cin, cout), lambda i: (0, 0)),
            pl.BlockSpec((1, cout), lambda i: (0, 0)),
        ],
        out_specs=pl.BlockSpec((LIN_ROWBLK, cout), lambda i: (i, 0)),
        out_shape=jax.ShapeDtypeStruct((h.shape[0], cout), jnp.float32),
    )(h, w, b.reshape(1, cout))


def _mlp_body(u_ref, w1_ref, b1_ref, w2_ref, b2_ref, o_ref):
    t = (
        jnp.dot(u_ref[...], w1_ref[...], preferred_element_type=jnp.float32,
                precision=lax.Precision.HIGHEST)
        + b1_ref[...]
    )
    t = jnp.where(t >= 0, t, 0.01 * t)
    o_ref[...] = (
        jnp.dot(t, w2_ref[...], preferred_element_type=jnp.float32,
                precision=lax.Precision.HIGHEST)
        + b2_ref[...]
    )


def _final_mlp(u, w1, b1, w2, b2):
    rows = u.shape[0]
    cout = w2.shape[1]
    return pl.pallas_call(
        _mlp_body,
        grid=(rows // LIN_ROWBLK,),
        in_specs=[
            pl.BlockSpec((LIN_ROWBLK, FEAT), lambda i: (i, 0)),
            pl.BlockSpec((FEAT, FEAT), lambda i: (0, 0)),
            pl.BlockSpec((1, FEAT), lambda i: (0, 0)),
            pl.BlockSpec((FEAT, cout), lambda i: (0, 0)),
            pl.BlockSpec((1, cout), lambda i: (0, 0)),
        ],
        out_specs=pl.BlockSpec((LIN_ROWBLK, cout), lambda i: (i, 0)),
        out_shape=jax.ShapeDtypeStruct((rows, cout), jnp.float32),
    )(u, w1, b1.reshape(1, FEAT), w2, b2.reshape(1, cout))


def _gather_max(btab, nbr):
    return jnp.max(btab[nbr], axis=1)


def _edge_conv(h, nbr, w, b, slope=0.2):
    cin = w.shape[0] // 2
    cout = w.shape[1]
    wcat = jnp.concatenate([w[:cin] - w[cin:], w[cin:]], axis=1)
    if h.shape[1] > cin:
        wcat = jnp.pad(wcat, ((0, h.shape[1] - cin), (0, 0)))
    bcat = jnp.concatenate([b, jnp.zeros_like(b)])
    ab = _lin(h, wcat, bcat)
    a, bt = ab[:, :cout], ab[:, cout:]
    m = _gather_max(bt, nbr)
    z = a + m
    return jnp.where(z >= 0, z, slope * z)


def kernel(x, W_emb, b_emb, W_blk, b_blk, W_sh, b_sh, W_r1, b_r1, W_r2, b_r2):
    pos_pad = jnp.zeros((ROWPAD, 8), jnp.float32).at[:NPTS, :3].set(x)

    dist = _pairwise_dist(pos_pad)
    _, idx = lax.top_k(-dist[:NPTS], KSEL)
    nbrs = idx[:, 1:]
    nbrs = jnp.pad(nbrs, ((0, ROWPAD - NPTS), (0, 0)))
    nbr_d = [nbrs[:, ::d][:, :KNB] for d in DILS]

    h = _edge_conv(pos_pad, nbr_d[0], W_emb, b_emb)
    for i in range(NBLOCKS):
        h_in = h
        acc = jnp.zeros_like(h)
        for j in range(len(DILS)):
            t = h_in
            for l in range(NLAYERS):
                t = _edge_conv(t, nbr_d[j], W_blk[i, j, l], b_blk[i, j, l])
            acc = acc + t
        h = h_in + acc / float(len(DILS))

    t = _edge_conv(h, nbr_d[0], W_sh, b_sh)
    up = t.reshape(ROWPAD * UPR, FEAT)
    q = _final_mlp(up, W_r1, b_r1, W_r2, b_r2)
    return q[: NPTS * UPR]


# speedup vs baseline: 1.0705x; 1.0705x over previous
import functools

import jax
import jax.numpy as jnp
from jax import lax
from jax.experimental import pallas as pl
from jax.experimental.pallas import tpu as pltpu
from jax.experimental.pallas import tpu_sc as plsc

NPTS = 10000
FEAT = 128
KNB = 16
DILS = (1, 2)
UPR = 4
NBLOCKS = 2
NLAYERS = 2
KSEL = KNB * max(DILS) + 1

ROWPAD = 10240
COLPAD = 10112
DIST_ROWBLK = 256
LIN_ROWBLK = 512


def _dist_body(pos_ref, post_ref, out_ref):
    pr = pos_ref[...]
    pt = post_ref[...]
    d2r = jnp.sum(pr * pr, axis=1, keepdims=True)
    d2c = jnp.sum(pt * pt, axis=0, keepdims=True)
    g = jnp.dot(pr, pt, preferred_element_type=jnp.float32)
    d = d2r + d2c - 2.0 * g
    col = lax.broadcasted_iota(jnp.int32, d.shape, 1)
    out_ref[...] = jnp.where(col < NPTS, d, jnp.inf)


def _pairwise_dist(pos_pad):
    post = pos_pad.T[:, :COLPAD]
    return pl.pallas_call(
        _dist_body,
        grid=(ROWPAD // DIST_ROWBLK,),
        in_specs=[
            pl.BlockSpec((DIST_ROWBLK, 8), lambda i: (i, 0)),
            pl.BlockSpec((8, COLPAD), lambda i: (0, 0)),
        ],
        out_specs=pl.BlockSpec((DIST_ROWBLK, COLPAD), lambda i: (i, 0)),
        out_shape=jax.ShapeDtypeStruct((ROWPAD, COLPAD), jnp.float32),
    )(pos_pad, post)


def _lin_body(h_ref, w_ref, b_ref, o_ref):
    o_ref[...] = (
        jnp.dot(h_ref[...], w_ref[...], preferred_element_type=jnp.float32,
                precision=lax.Precision.HIGHEST)
        + b_ref[...]
    )


def _lin(h, w, b):
    cin, cout = w.shape
    return pl.pallas_call(
        _lin_body,
        grid=(h.shape[0] // LIN_ROWBLK,),
        in_specs=[
            pl.BlockSpec((LIN_ROWBLK, cin), lambda i: (i, 0)),
            pl.BlockSpec((cin, cout), lambda i: (0, 0)),
            pl.BlockSpec((1, cout), lambda i: (0, 0)),
        ],
        out_specs=pl.BlockSpec((LIN_ROWBLK, cout), lambda i: (i, 0)),
        out_shape=jax.ShapeDtypeStruct((h.shape[0], cout), jnp.float32),
    )(h, w, b.reshape(1, cout))


def _mlp_body(u_ref, w1_ref, b1_ref, w2_ref, b2_ref, o_ref):
    t = (
        jnp.dot(u_ref[...], w1_ref[...], preferred_element_type=jnp.float32,
                precision=lax.Precision.HIGHEST)
        + b1_ref[...]
    )
    t = jnp.where(t >= 0, t, 0.01 * t)
    o_ref[...] = (
        jnp.dot(t, w2_ref[...], preferred_element_type=jnp.float32,
                precision=lax.Precision.HIGHEST)
        + b2_ref[...]
    )


def _final_mlp(u, w1, b1, w2, b2):
    rows = u.shape[0]
    cout = w2.shape[1]
    return pl.pallas_call(
        _mlp_body,
        grid=(rows // LIN_ROWBLK,),
        in_specs=[
            pl.BlockSpec((LIN_ROWBLK, FEAT), lambda i: (i, 0)),
            pl.BlockSpec((FEAT, FEAT), lambda i: (0, 0)),
            pl.BlockSpec((1, FEAT), lambda i: (0, 0)),
            pl.BlockSpec((FEAT, cout), lambda i: (0, 0)),
            pl.BlockSpec((1, cout), lambda i: (0, 0)),
        ],
        out_specs=pl.BlockSpec((LIN_ROWBLK, cout), lambda i: (i, 0)),
        out_shape=jax.ShapeDtypeStruct((rows, cout), jnp.float32),
    )(u, w1, b1.reshape(1, FEAT), w2, b2.reshape(1, cout))


NTILES = 32
RPT = ROWPAD // NTILES


def _gmax_body(cout, GCH, btab_hbm, nbr_hbm, out_hbm, idx_v, rows0, rows1,
               oc_v, sem0, sem1):
    wid = lax.axis_index("s") * 2 + lax.axis_index("c")
    base = wid * RPT
    pltpu.sync_copy(nbr_hbm.at[pl.ds(base * KNB, RPT * KNB)], idx_v)

    nch = RPT // GCH
    ncc = cout // 16

    def gather(g, rows, sem):
        pltpu.async_copy(
            btab_hbm.at[idx_v.at[pl.ds(g * (GCH * KNB), GCH * KNB)]], rows,
            sem)

    def drain(rows, sem):
        pltpu.make_async_copy(btab_hbm.at[pl.ds(0, GCH * KNB)], rows,
                              sem).wait()

    def compute(g, rows, sem):
        drain(rows, sem)
        for j in range(GCH):

            def col_body(c, _, j=j):
                cof = c * 16
                acc = rows[j * KNB, pl.ds(cof, 16)]
                for t in range(1, KNB):
                    acc = jnp.maximum(acc, rows[j * KNB + t, pl.ds(cof, 16)])
                oc_v[j, pl.ds(cof, 16)] = acc
                return 0

            lax.fori_loop(0, ncc, col_body, 0)
        pltpu.sync_copy(oc_v, out_hbm.at[pl.ds(base + g * GCH, GCH)])

    gather(0, rows0, sem0)
    gather(1, rows1, sem1)

    def step(g2, _):
        g = g2 * 2
        compute(g, rows0, sem0)
        pl.when(g + 2 < nch)(lambda: gather(g + 2, rows0, sem0))
        compute(g + 1, rows1, sem1)
        pl.when(g + 3 < nch)(lambda: gather(g + 3, rows1, sem1))
        return 0

    lax.fori_loop(0, nch // 2, step, 0)


def _gather_max(btab, nbr, cout):
    GCH = 8 if cout <= 128 else 4
    mesh = plsc.VectorSubcoreMesh(core_axis_name="c", subcore_axis_name="s")
    kfn = functools.partial(
        pl.kernel,
        mesh=mesh,
        out_type=jax.ShapeDtypeStruct((ROWPAD, cout), jnp.float32),
        scratch_types=[
            pltpu.VMEM((RPT * KNB,), jnp.int32),
            pltpu.VMEM((GCH * KNB, cout), jnp.float32),
            pltpu.VMEM((GCH * KNB, cout), jnp.float32),
            pltpu.VMEM((GCH, cout), jnp.float32),
            pltpu.SemaphoreType.DMA,
            pltpu.SemaphoreType.DMA,
        ],
    )(functools.partial(_gmax_body, cout, GCH))
    return kfn(btab, nbr.reshape(-1))


def _edge_conv(h, nbr, w, b, slope=0.2):
    cin = w.shape[0] // 2
    cout = w.shape[1]
    wcat = jnp.concatenate([w[:cin] - w[cin:], w[cin:]], axis=1)
    if h.shape[1] > cin:
        wcat = jnp.pad(wcat, ((0, h.shape[1] - cin), (0, 0)))
    bcat = jnp.concatenate([b, jnp.zeros_like(b)])
    ab = _lin(h, wcat, bcat)
    a, bt = ab[:, :cout], ab[:, cout:]
    m = _gather_max(bt, nbr, cout)
    z = a + m
    return jnp.where(z >= 0, z, slope * z)


def kernel(x, W_emb, b_emb, W_blk, b_blk, W_sh, b_sh, W_r1, b_r1, W_r2, b_r2):
    pos_pad = jnp.zeros((ROWPAD, 8), jnp.float32).at[:NPTS, :3].set(x)

    dist = _pairwise_dist(pos_pad)
    _, idx = lax.top_k(-dist[:NPTS], KSEL)
    nbrs = idx[:, 1:]
    nbrs = jnp.pad(nbrs, ((0, ROWPAD - NPTS), (0, 0)))
    rid = lax.broadcasted_iota(jnp.int32, nbrs.shape, 0)
    nbrs = jnp.where(rid < NPTS, nbrs, rid)
    nbr_d = [nbrs[:, ::d][:, :KNB] for d in DILS]

    h = _edge_conv(pos_pad, nbr_d[0], W_emb, b_emb)
    for i in range(NBLOCKS):
        h_in = h
        acc = jnp.zeros_like(h)
        for j in range(len(DILS)):
            t = h_in
            for l in range(NLAYERS):
                t = _edge_conv(t, nbr_d[j], W_blk[i, j, l], b_blk[i, j, l])
            acc = acc + t
        h = h_in + acc / float(len(DILS))

    t = _edge_conv(h, nbr_d[0], W_sh, b_sh)
    up = t.reshape(ROWPAD * UPR, FEAT)
    q = _final_mlp(up, W_r1, b_r1, W_r2, b_r2)
    return q[: NPTS * UPR]


# speedup vs baseline: 1.0710x; 1.0005x over previous
import functools

import jax
import jax.numpy as jnp
from jax import lax
from jax.experimental import pallas as pl
from jax.experimental.pallas import tpu as pltpu
from jax.experimental.pallas import tpu_sc as plsc

NPTS = 10000
FEAT = 128
KNB = 16
DILS = (1, 2)
UPR = 4
NBLOCKS = 2
NLAYERS = 2
KSEL = KNB * max(DILS) + 1

ROWPAD = 10240
COLPAD = 10112
DIST_ROWBLK = 256
LIN_ROWBLK = 512


def _dist_body(pos_ref, post_ref, out_ref):
    pr = pos_ref[...]
    pt = post_ref[...]
    d2r = jnp.sum(pr * pr, axis=1, keepdims=True)
    d2c = jnp.sum(pt * pt, axis=0, keepdims=True)
    g = jnp.dot(pr, pt, preferred_element_type=jnp.float32)
    d = d2r + d2c - 2.0 * g
    col = lax.broadcasted_iota(jnp.int32, d.shape, 1)
    out_ref[...] = jnp.where(col < NPTS, d, jnp.inf)


def _pairwise_dist(pos_pad):
    post = pos_pad.T[:, :COLPAD]
    return pl.pallas_call(
        _dist_body,
        grid=(ROWPAD // DIST_ROWBLK,),
        in_specs=[
            pl.BlockSpec((DIST_ROWBLK, 8), lambda i: (i, 0)),
            pl.BlockSpec((8, COLPAD), lambda i: (0, 0)),
        ],
        out_specs=pl.BlockSpec((DIST_ROWBLK, COLPAD), lambda i: (i, 0)),
        out_shape=jax.ShapeDtypeStruct((ROWPAD, COLPAD), jnp.float32),
    )(pos_pad, post)


def _lin_body(h_ref, w_ref, b_ref, o_ref):
    o_ref[...] = (
        jnp.dot(h_ref[...], w_ref[...], preferred_element_type=jnp.float32,
                precision=lax.Precision.HIGHEST)
        + b_ref[...]
    )


def _lin(h, w, b):
    cin, cout = w.shape
    return pl.pallas_call(
        _lin_body,
        grid=(h.shape[0] // LIN_ROWBLK,),
        in_specs=[
            pl.BlockSpec((LIN_ROWBLK, cin), lambda i: (i, 0)),
            pl.BlockSpec((cin, cout), lambda i: (0, 0)),
            pl.BlockSpec((1, cout), lambda i: (0, 0)),
        ],
        out_specs=pl.BlockSpec((LIN_ROWBLK, cout), lambda i: (i, 0)),
        out_shape=jax.ShapeDtypeStruct((h.shape[0], cout), jnp.float32),
    )(h, w, b.reshape(1, cout))


def _mlp_body(u_ref, w1_ref, b1_ref, w2_ref, b2_ref, o_ref):
    t = (
        jnp.dot(u_ref[...], w1_ref[...], preferred_element_type=jnp.float32,
                precision=lax.Precision.HIGHEST)
        + b1_ref[...]
    )
    t = jnp.where(t >= 0, t, 0.01 * t)
    o_ref[...] = (
        jnp.dot(t, w2_ref[...], preferred_element_type=jnp.float32,
                precision=lax.Precision.HIGHEST)
        + b2_ref[...]
    )


def _final_mlp(u, w1, b1, w2, b2):
    rows = u.shape[0]
    cout = w2.shape[1]
    return pl.pallas_call(
        _mlp_body,
        grid=(rows // LIN_ROWBLK,),
        in_specs=[
            pl.BlockSpec((LIN_ROWBLK, FEAT), lambda i: (i, 0)),
            pl.BlockSpec((FEAT, FEAT), lambda i: (0, 0)),
            pl.BlockSpec((1, FEAT), lambda i: (0, 0)),
            pl.BlockSpec((FEAT, cout), lambda i: (0, 0)),
            pl.BlockSpec((1, cout), lambda i: (0, 0)),
        ],
        out_specs=pl.BlockSpec((LIN_ROWBLK, cout), lambda i: (i, 0)),
        out_shape=jax.ShapeDtypeStruct((rows, cout), jnp.float32),
    )(u, w1, b1.reshape(1, FEAT), w2, b2.reshape(1, cout))


NTILES = 32
RPT = ROWPAD // NTILES


def _gmax_body(cout, GCH, btab_hbm, nbr_hbm, out_hbm, idx_v, rows0, rows1,
               oc_v, sem0, sem1):
    wid = lax.axis_index("s") * 2 + lax.axis_index("c")
    base = wid * RPT
    pltpu.sync_copy(nbr_hbm.at[pl.ds(base * KNB, RPT * KNB)], idx_v)

    nch = RPT // GCH
    ncc = cout // 16

    def gather(g, rows, sem):
        pltpu.async_copy(
            btab_hbm.at[idx_v.at[pl.ds(g * (GCH * KNB), GCH * KNB)]], rows,
            sem)

    def drain(rows, sem):
        pltpu.make_async_copy(btab_hbm.at[pl.ds(0, GCH * KNB)], rows,
                              sem).wait()

    def compute(g, rows, sem):
        drain(rows, sem)
        for j in range(GCH):

            def col_body(c, _, j=j):
                cof = c * 16
                acc = rows[j * KNB, pl.ds(cof, 16)]
                for t in range(1, KNB):
                    acc = jnp.maximum(acc, rows[j * KNB + t, pl.ds(cof, 16)])
                oc_v[j, pl.ds(cof, 16)] = acc
                return 0

            lax.fori_loop(0, ncc, col_body, 0)
        pltpu.sync_copy(oc_v, out_hbm.at[pl.ds(base + g * GCH, GCH)])

    gather(0, rows0, sem0)
    gather(1, rows1, sem1)

    def step(g2, _):
        g = g2 * 2
        compute(g, rows0, sem0)
        pl.when(g + 2 < nch)(lambda: gather(g + 2, rows0, sem0))
        compute(g + 1, rows1, sem1)
        pl.when(g + 3 < nch)(lambda: gather(g + 3, rows1, sem1))
        return 0

    lax.fori_loop(0, nch // 2, step, 0)


def _gather_max(btab, nbr, cout):
    GCH = 8 if cout <= 128 else 4
    mesh = plsc.VectorSubcoreMesh(core_axis_name="c", subcore_axis_name="s")
    kfn = functools.partial(
        pl.kernel,
        mesh=mesh,
        out_type=jax.ShapeDtypeStruct((ROWPAD, cout), jnp.float32),
        scratch_types=[
            pltpu.VMEM((RPT * KNB,), jnp.int32),
            pltpu.VMEM((GCH * KNB, cout), jnp.float32),
            pltpu.VMEM((GCH * KNB, cout), jnp.float32),
            pltpu.VMEM((GCH, cout), jnp.float32),
            pltpu.SemaphoreType.DMA,
            pltpu.SemaphoreType.DMA,
        ],
    )(functools.partial(_gmax_body, cout, GCH))
    return kfn(btab, nbr.reshape(-1))


def _edge_conv(h, nbr, w, b, slope=0.2):
    cin = w.shape[0] // 2
    cout = w.shape[1]
    wcat = jnp.concatenate([w[:cin] - w[cin:], w[cin:]], axis=1)
    if h.shape[1] > cin:
        wcat = jnp.pad(wcat, ((0, h.shape[1] - cin), (0, 0)))
    bcat = jnp.concatenate([b, jnp.zeros_like(b)])
    ab = _lin(h, wcat, bcat)
    a, bt = ab[:, :cout], ab[:, cout:]
    m = _gather_max(bt, nbr, cout)
    z = a + m
    return jnp.where(z >= 0, z, slope * z)


def kernel(x, W_emb, b_emb, W_blk, b_blk, W_sh, b_sh, W_r1, b_r1, W_r2, b_r2):
    pos_pad = jnp.zeros((ROWPAD, 8), jnp.float32).at[:NPTS, :3].set(x)

    dist = _pairwise_dist(pos_pad)
    _, idx = lax.top_k(-dist[:NPTS], KSEL)
    nbrs = jnp.pad(idx[:, 1:], ((0, ROWPAD - NPTS), (0, 0)))
    rid = lax.broadcasted_iota(jnp.int32, nbrs.shape, 0)
    nbrs = jnp.where(rid < NPTS, nbrs, rid)
    nbr_d = [nbrs[:, ::d][:, :KNB] for d in DILS]

    h = _edge_conv(pos_pad, nbr_d[0], W_emb, b_emb)
    for i in range(NBLOCKS):
        h_in = h
        acc = jnp.zeros_like(h)
        for j in range(len(DILS)):
            t = h_in
            for l in range(NLAYERS):
                t = _edge_conv(t, nbr_d[j], W_blk[i, j, l], b_blk[i, j, l])
            acc = acc + t
        h = h_in + acc / float(len(DILS))

    t = _edge_conv(h, nbr_d[0], W_sh, b_sh)
    up = t.reshape(ROWPAD * UPR, FEAT)
    q = _final_mlp(up, W_r1, b_r1, W_r2, b_r2)
    return q[: NPTS * UPR]
